# baseline (speedup 1.0000x reference)
import functools

import numpy as np

import jax
import jax.numpy as jnp
from jax import lax
from jax.experimental import pallas as pl
from jax.experimental.pallas import tpu as pltpu
from jax.experimental.pallas import tpu_sc as plsc

_NC = 2
_NS = 16
_NW = _NC * _NS
_CH = 64
_LANES = 16


def _pack_perm(d):
    perm = np.empty(d, dtype=np.int32)
    for blk in range(d // 32):
        base = blk * 32
        for i in range(16):
            perm[base + 2 * i] = base + i
            perm[base + 2 * i + 1] = base + 16 + i
    inv = np.empty_like(perm)
    inv[perm] = np.arange(d, dtype=np.int32)
    return perm, inv


def _sc_gather(k2, v2, flat_idx, d):
    R = flat_idx.shape[0]
    rpw = R // _NW
    n_chunks = rpw // _CH

    mesh = plsc.VectorSubcoreMesh(
        core_axis_name="c", subcore_axis_name="s",
        num_cores=_NC, num_subcores=_NS)

    @functools.partial(
        pl.kernel,
        mesh=mesh,
        compiler_params=pltpu.CompilerParams(needs_layout_passes=False),
        out_type=(
            jax.ShapeDtypeStruct((R, d // 2), jnp.int32),
            jax.ShapeDtypeStruct((R, d // 2), jnp.int32),
        ),
        scratch_types=[
            pltpu.VMEM((rpw,), jnp.int32),
            pltpu.VMEM((2, _CH, d), jnp.float32),
            pltpu.VMEM((2, _CH, d), jnp.float32),
            pltpu.VMEM((2, _CH, d // 2), jnp.int32),
            pltpu.VMEM((2, _CH, d // 2), jnp.int32),
            pltpu.SemaphoreType.DMA,
            pltpu.SemaphoreType.DMA,
            pltpu.SemaphoreType.DMA,
            pltpu.SemaphoreType.DMA,
            pltpu.SemaphoreType.DMA,
            pltpu.SemaphoreType.DMA,
            pltpu.SemaphoreType.DMA,
            pltpu.SemaphoreType.DMA,
        ],
    )
    def gather_kernel(k_hbm, v_hbm, idx_hbm, ko_hbm, vo_hbm,
                      idx_v, kbuf, vbuf, kbf, vbf,
                      ksem0, ksem1, vsem0, vsem1,
                      kwsem0, kwsem1, vwsem0, vwsem1):
        wid = lax.axis_index("s") * _NC + lax.axis_index("c")
        base = wid * rpw
        pltpu.sync_copy(idx_hbm.at[pl.ds(base, rpw)], idx_v)
        ksems = (ksem0, ksem1)
        vsems = (vsem0, vsem1)
        kwsems = (kwsem0, kwsem1)
        vwsems = (vwsem0, vwsem1)

        def start(c):
            p = c % 2
            ix = idx_v.at[pl.ds(c * _CH, _CH)]
            ck = pltpu.async_copy(k_hbm.at[ix], kbuf.at[p], ksems[p])
            cv = pltpu.async_copy(v_hbm.at[ix], vbuf.at[p], vsems[p])
            return ck, cv

        def pack_rows(src, dst, p):
            p_vec = jnp.full((_LANES,), p, jnp.int32)
            ci = lax.iota(jnp.int32, _LANES)

            def row(r, carry):
                r_vec = jnp.zeros((_LANES,), jnp.int32) + r
                for blk in range(d // 32):
                    a = plsc.load_gather(src, [p_vec, r_vec, ci + blk * 32])
                    b = plsc.load_gather(
                        src, [p_vec, r_vec, ci + blk * 32 + _LANES])
                    w = plsc.pack(a, b, format=plsc.PackFormat.INTERLEAVED)
                    wi = plsc.bitcast(w, jnp.int32)
                    plsc.store_scatter(dst, [p_vec, r_vec, ci + blk * _LANES],
                                       wi)
                return carry
            lax.fori_loop(0, _CH, row, 0)

        inflight = [start(0), start(1)]
        wb = [None, None]
        for c in range(n_chunks):
            p = c % 2
            ck, cv = inflight[p]
            ck.wait()
            cv.wait()
            if wb[p] is not None:
                for w in wb[p]:
                    w.wait()
            pack_rows(kbuf, kbf, p)
            pack_rows(vbuf, vbf, p)
            wk = pltpu.async_copy(
                kbf.at[p], ko_hbm.at[pl.ds(base + c * _CH, _CH)], kwsems[p])
            wv = pltpu.async_copy(
                vbf.at[p], vo_hbm.at[pl.ds(base + c * _CH, _CH)], vwsems[p])
            wb[p] = (wk, wv)
            if c + 2 < n_chunks:
                inflight[p] = start(c + 2)
        for pair in wb:
            if pair is not None:
                for w in pair:
                    w.wait()

    return gather_kernel(k2, v2, flat_idx)


def _attn_body(q_ref, k_ref, v_ref, o_ref, *, bg, sm_scale):
    ss = []
    for i in range(bg):
        qi = (q_ref[i] * sm_scale).astype(jnp.bfloat16)
        ss.append(jax.lax.dot_general(
            qi, k_ref[i], (((1,), (1,)), ((), ())),
            preferred_element_type=jnp.float32))
    aa = []
    for s in ss:
        m = jnp.max(s, axis=1, keepdims=True)
        e = jnp.exp(s - m)
        l = jnp.sum(e, axis=1, keepdims=True)
        aa.append((e / l).astype(jnp.bfloat16))
    for i in range(bg):
        o_ref[i] = jax.lax.dot_general(
            aa[i], v_ref[i], (((1,), (0,)), ((), ())),
            preferred_element_type=jnp.float32)


def _tc_attn(q_r, k_sel, v_sel, sm_scale):
    NG, G, d = q_r.shape
    L = k_sel.shape[1]
    BG = 8
    grid = (NG // BG,)
    return pl.pallas_call(
        functools.partial(_attn_body, bg=BG, sm_scale=sm_scale),
        grid=grid,
        in_specs=[
            pl.BlockSpec((BG, G, d), lambda i: (i, 0, 0)),
            pl.BlockSpec((BG, L, d), lambda i: (i, 0, 0)),
            pl.BlockSpec((BG, L, d), lambda i: (i, 0, 0)),
        ],
        out_specs=pl.BlockSpec((BG, G, d), lambda i: (i, 0, 0)),
        out_shape=jax.ShapeDtypeStruct((NG, G, d), jnp.float32),
    )(q_r, k_sel, v_sel)


def kernel(q, k, v, sparse_indices):
    B, S1, N1, d = q.shape
    _, S2, N2, _ = k.shape
    L = sparse_indices.shape[-1]
    G = N1 // N2
    R = B * S1 * N2 * L
    sm_scale = float(d) ** -0.5
    perm, inv = _pack_perm(d)

    b_ix = jnp.arange(B, dtype=jnp.int32).reshape(B, 1, 1, 1)
    h_ix = jnp.arange(N2, dtype=jnp.int32).reshape(1, 1, N2, 1)
    flat = ((b_ix * S2 + sparse_indices.astype(jnp.int32)) * N2 + h_ix)
    flat = flat.reshape(R)

    k2 = k.reshape(B * S2 * N2, d)
    v2 = v.reshape(B * S2 * N2, d)

    k_sel_w, v_sel_w = _sc_gather(k2, v2, flat, d)
    k_sel = lax.bitcast_convert_type(k_sel_w, jnp.bfloat16).reshape(R, d)
    v_sel = lax.bitcast_convert_type(v_sel_w, jnp.bfloat16).reshape(R, d)

    q_r = q.reshape(B * S1 * N2, G, d)[:, :, perm]
    out = _tc_attn(q_r, k_sel.reshape(-1, L, d), v_sel.reshape(-1, L, d),
                   sm_scale)
    out = out[:, :, inv]
    return out.reshape(B, S1, N1, d).astype(jnp.float16)


# speedup vs baseline: 2.5818x; 2.5818x over previous
import functools

import numpy as np

import jax
import jax.numpy as jnp
from jax import lax
from jax.experimental import pallas as pl
from jax.experimental.pallas import tpu as pltpu
from jax.experimental.pallas import tpu_sc as plsc

_NC = 2
_NS = 16
_NW = _NC * _NS
_CH = 128
_LANES = 16


def _pack_perm(d):
    perm = np.empty(d, dtype=np.int32)
    for blk in range(d // 32):
        base = blk * 32
        for i in range(16):
            perm[base + 2 * i] = base + i
            perm[base + 2 * i + 1] = base + 16 + i
    inv = np.empty_like(perm)
    inv[perm] = np.arange(d, dtype=np.int32)
    return perm, inv


def _sc_gather(k2, v2, flat_idx, d):
    R = flat_idx.shape[0]
    rpw = R // _NW
    n_chunks = rpw // _CH

    mesh = plsc.VectorSubcoreMesh(
        core_axis_name="c", subcore_axis_name="s",
        num_cores=_NC, num_subcores=_NS)

    @functools.partial(
        pl.kernel,
        mesh=mesh,
        compiler_params=pltpu.CompilerParams(needs_layout_passes=False),
        out_type=(
            jax.ShapeDtypeStruct((R // 2, d), jnp.int32),
            jax.ShapeDtypeStruct((R // 2, d), jnp.int32),
        ),
        scratch_types=[
            pltpu.VMEM((2, _CH), jnp.int32),
            pltpu.VMEM((2, _CH, d), jnp.float32),
            pltpu.VMEM((2, _CH, d), jnp.float32),
            pltpu.VMEM((_CH // 2, d), jnp.int32),
            pltpu.VMEM((_CH // 2, d), jnp.int32),
            pltpu.SemaphoreType.DMA,
            pltpu.SemaphoreType.DMA,
            pltpu.SemaphoreType.DMA,
            pltpu.SemaphoreType.DMA,
            pltpu.SemaphoreType.DMA,
            pltpu.SemaphoreType.DMA,
            pltpu.SemaphoreType.DMA,
            pltpu.SemaphoreType.DMA,
        ],
    )
    def gather_kernel(k_hbm, v_hbm, idx_hbm, ko_hbm, vo_hbm,
                      idx_v, kbuf, vbuf, kdst, vdst,
                      ksem0, ksem1, vsem0, vsem1,
                      isem0, isem1, kwsem, vwsem):
        wid = lax.axis_index("s") * _NC + lax.axis_index("c")
        base = wid * rpw
        ksems = (ksem0, ksem1)
        vsems = (vsem0, vsem1)
        isems = (isem0, isem1)

        def load_idx(c):
            p = c % 2
            off = pl.multiple_of(base + c * _CH, _CH)
            return pltpu.async_copy(
                idx_hbm.at[pl.ds(off, _CH)], idx_v.at[p], isems[p])

        def start(c):
            p = c % 2
            ix = idx_v.at[p]
            ck = pltpu.async_copy(k_hbm.at[ix], kbuf.at[p], ksems[p])
            cv = pltpu.async_copy(v_hbm.at[ix], vbuf.at[p], vsems[p])
            return ck, cv

        p0_vec = jnp.full((_LANES,), 0, jnp.int32)
        p1_vec = jnp.full((_LANES,), 1, jnp.int32)
        ci = lax.iota(jnp.int32, _LANES)
        _RU = 2

        def pack_chunk(p):
            p_vec = p0_vec if p == 0 else p1_vec

            def rows(i, carry):
                for j in range(_RU):
                    r2 = i * _RU + j
                    r2_vec = jnp.zeros((_LANES,), jnp.int32) + r2
                    ra_vec = r2_vec * 2
                    rb_vec = ra_vec + 1
                    for blk in range(d // _LANES):
                        col = ci + blk * _LANES
                        for src, dst in ((kbuf, kdst), (vbuf, vdst)):
                            a = plsc.load_gather(src, [p_vec, ra_vec, col])
                            b = plsc.load_gather(src, [p_vec, rb_vec, col])
                            w = plsc.pack(
                                a, b, format=plsc.PackFormat.INTERLEAVED)
                            wi = plsc.bitcast(w, jnp.int32)
                            plsc.store_scatter(dst, [r2_vec, col], wi)
                return carry
            lax.fori_loop(0, (_CH // 2) // _RU, rows, 0)

        load_idx(0).wait()
        i1 = load_idx(1)
        inflight = [start(0), None]
        i1.wait()
        inflight[1] = start(1)
        wb = None
        for c in range(n_chunks):
            p = c % 2
            ck, cv = inflight[p]
            ck.wait()
            cv.wait()
            nxt = None
            if c + 2 < n_chunks:
                nxt = load_idx(c + 2)
            if wb is not None:
                for w in wb:
                    w.wait()
            pack_chunk(p)
            woff = pl.multiple_of((base + c * _CH) // 2, _CH // 2)
            wk = pltpu.async_copy(
                kdst, ko_hbm.at[pl.ds(woff, _CH // 2)], kwsem)
            wv = pltpu.async_copy(
                vdst, vo_hbm.at[pl.ds(woff, _CH // 2)], vwsem)
            wb = (wk, wv)
            if nxt is not None:
                nxt.wait()
                inflight[p] = start(c + 2)
        for w in wb:
            w.wait()

    return gather_kernel(k2, v2, flat_idx)


def _attn_body(q_ref, k_ref, v_ref, o_ref, *, bg, sm_scale):
    ss = []
    vs = []
    for i in range(bg):
        qi = (q_ref[i] * sm_scale).astype(jnp.bfloat16)
        ki = pltpu.bitcast(k_ref[i], jnp.bfloat16)
        vs.append(pltpu.bitcast(v_ref[i], jnp.bfloat16))
        ss.append(jax.lax.dot_general(
            qi, ki, (((1,), (1,)), ((), ())),
            preferred_element_type=jnp.float32))
    aa = []
    for s in ss:
        m = jnp.max(s, axis=1, keepdims=True)
        e = jnp.exp(s - m)
        l = jnp.sum(e, axis=1, keepdims=True)
        aa.append((e / l).astype(jnp.bfloat16))
    for i in range(bg):
        o_ref[i] = jax.lax.dot_general(
            aa[i], vs[i], (((1,), (0,)), ((), ())),
            preferred_element_type=jnp.float32)


def _tc_attn(q_r, k_sel_w, v_sel_w, sm_scale):
    NG, G, d = q_r.shape
    Lw = k_sel_w.shape[1]
    BG = 8
    grid = (NG // BG,)
    return pl.pallas_call(
        functools.partial(_attn_body, bg=BG, sm_scale=sm_scale),
        grid=grid,
        in_specs=[
            pl.BlockSpec((BG, G, d), lambda i: (i, 0, 0)),
            pl.BlockSpec((BG, Lw, d), lambda i: (i, 0, 0)),
            pl.BlockSpec((BG, Lw, d), lambda i: (i, 0, 0)),
        ],
        out_specs=pl.BlockSpec((BG, G, d), lambda i: (i, 0, 0)),
        out_shape=jax.ShapeDtypeStruct((NG, G, d), jnp.float32),
    )(q_r, k_sel_w, v_sel_w)


def kernel(q, k, v, sparse_indices):
    B, S1, N1, d = q.shape
    _, S2, N2, _ = k.shape
    L = sparse_indices.shape[-1]
    G = N1 // N2
    R = B * S1 * N2 * L
    sm_scale = float(d) ** -0.5

    b_ix = jnp.arange(B, dtype=jnp.int32).reshape(B, 1, 1, 1)
    h_ix = jnp.arange(N2, dtype=jnp.int32).reshape(1, 1, N2, 1)
    flat = ((b_ix * S2 + sparse_indices.astype(jnp.int32)) * N2 + h_ix)
    flat = flat.reshape(R)

    k2 = k.reshape(B * S2 * N2, d)
    v2 = v.reshape(B * S2 * N2, d)

    k_sel_w, v_sel_w = _sc_gather(k2, v2, flat, d)

    q_r = q.reshape(B * S1 * N2, G, d)
    out = _tc_attn(q_r, k_sel_w.reshape(-1, L // 2, d),
                   v_sel_w.reshape(-1, L // 2, d), sm_scale)
    return out.reshape(B, S1, N1, d).astype(jnp.float16)


# speedup vs baseline: 3.8309x; 1.4838x over previous
import functools

import jax
import jax.numpy as jnp
from jax import lax
from jax.experimental import pallas as pl
from jax.experimental.pallas import tpu as pltpu
from jax.experimental.pallas import tpu_sc as plsc

_NC = 2
_NS = 16
_NW = _NC * _NS
_CH = 128


def _sc_gather(k2, v2, flat_idx, d):
    R = flat_idx.shape[0]
    rpw = R // _NW
    n_chunks = rpw // _CH

    mesh = plsc.VectorSubcoreMesh(
        core_axis_name="c", subcore_axis_name="s",
        num_cores=_NC, num_subcores=_NS)

    @functools.partial(
        pl.kernel,
        mesh=mesh,
        out_type=(
            jax.ShapeDtypeStruct((R, d), jnp.float32),
            jax.ShapeDtypeStruct((R, d), jnp.float32),
        ),
        scratch_types=[
            pltpu.VMEM((rpw,), jnp.int32),
            pltpu.VMEM((2, _CH, d), jnp.float32),
            pltpu.VMEM((2, _CH, d), jnp.float32),
            pltpu.SemaphoreType.DMA,
            pltpu.SemaphoreType.DMA,
            pltpu.SemaphoreType.DMA,
            pltpu.SemaphoreType.DMA,
        ],
    )
    def gather_kernel(k_hbm, v_hbm, idx_hbm, ko_hbm, vo_hbm,
                      idx_v, kbuf, vbuf, ksem0, ksem1, vsem0, vsem1):
        wid = lax.axis_index("s") * _NC + lax.axis_index("c")
        base = wid * rpw
        pltpu.sync_copy(idx_hbm.at[pl.ds(base, rpw)], idx_v)
        ksems = (ksem0, ksem1)
        vsems = (vsem0, vsem1)

        def start(c):
            p = c % 2
            ix = idx_v.at[pl.ds(c * _CH, _CH)]
            ck = pltpu.async_copy(k_hbm.at[ix], kbuf.at[p], ksems[p])
            cv = pltpu.async_copy(v_hbm.at[ix], vbuf.at[p], vsems[p])
            return ck, cv

        inflight = [start(0), start(1)]
        for c in range(n_chunks):
            p = c % 2
            ck, cv = inflight[p]
            ck.wait()
            cv.wait()
            pltpu.sync_copy(kbuf.at[p], ko_hbm.at[pl.ds(base + c * _CH, _CH)])
            pltpu.sync_copy(vbuf.at[p], vo_hbm.at[pl.ds(base + c * _CH, _CH)])
            if c + 2 < n_chunks:
                inflight[p] = start(c + 2)

    return gather_kernel(k2, v2, flat_idx)


def _attn_body(q_ref, k_ref, v_ref, o_ref, *, bg, sm_scale):
    ss = []
    for i in range(bg):
        qi = (q_ref[i] * sm_scale).astype(jnp.bfloat16)
        ki = k_ref[i].astype(jnp.bfloat16)
        ss.append(jax.lax.dot_general(
            qi, ki, (((1,), (1,)), ((), ())),
            preferred_element_type=jnp.float32))
    aa = []
    for s in ss:
        m = jnp.max(s, axis=1, keepdims=True)
        e = jnp.exp(s - m)
        l = jnp.sum(e, axis=1, keepdims=True)
        aa.append((e / l).astype(jnp.bfloat16))
    for i in range(bg):
        vi = v_ref[i].astype(jnp.bfloat16)
        o_ref[i] = jax.lax.dot_general(
            aa[i], vi, (((1,), (0,)), ((), ())),
            preferred_element_type=jnp.float32)


def _tc_attn(q_r, k_sel, v_sel, sm_scale):
    NG, G, d = q_r.shape
    L = k_sel.shape[1]
    BG = 8
    grid = (NG // BG,)
    return pl.pallas_call(
        functools.partial(_attn_body, bg=BG, sm_scale=sm_scale),
        grid=grid,
        in_specs=[
            pl.BlockSpec((BG, G, d), lambda i: (i, 0, 0)),
            pl.BlockSpec((BG, L, d), lambda i: (i, 0, 0)),
            pl.BlockSpec((BG, L, d), lambda i: (i, 0, 0)),
        ],
        out_specs=pl.BlockSpec((BG, G, d), lambda i: (i, 0, 0)),
        out_shape=jax.ShapeDtypeStruct((NG, G, d), jnp.float32),
    )(q_r, k_sel, v_sel)


def kernel(q, k, v, sparse_indices):
    B, S1, N1, d = q.shape
    _, S2, N2, _ = k.shape
    L = sparse_indices.shape[-1]
    G = N1 // N2
    R = B * S1 * N2 * L
    sm_scale = float(d) ** -0.5

    b_ix = jnp.arange(B, dtype=jnp.int32).reshape(B, 1, 1, 1)
    h_ix = jnp.arange(N2, dtype=jnp.int32).reshape(1, 1, N2, 1)
    flat = ((b_ix * S2 + sparse_indices.astype(jnp.int32)) * N2 + h_ix)
    flat = flat.reshape(R)

    k2 = k.reshape(B * S2 * N2, d)
    v2 = v.reshape(B * S2 * N2, d)

    q_r = q.reshape(B * S1 * N2, G, d)

    H = 2
    RH = R // H
    NGH = (B * S1 * N2) // H
    outs = []
    for h in range(H):
        k_sel, v_sel = _sc_gather(k2, v2, flat[h * RH:(h + 1) * RH], d)
        outs.append(_tc_attn(
            q_r[h * NGH:(h + 1) * NGH],
            k_sel.reshape(-1, L, d), v_sel.reshape(-1, L, d), sm_scale))
    out = jnp.concatenate(outs, axis=0)
    return out.reshape(B, S1, N1, d).astype(jnp.float16)


# speedup vs baseline: 4.4946x; 1.1732x over previous
import functools

import jax
import jax.numpy as jnp
from jax import lax
from jax.experimental import pallas as pl
from jax.experimental.pallas import tpu as pltpu
from jax.experimental.pallas import tpu_sc as plsc

_NC = 2
_NS = 16
_NW = _NC * _NS
_CH = 128


def _sc_gather(k2, v2, flat_idx, d):
    R = flat_idx.shape[0]
    rpw = R // _NW
    n_chunks = rpw // _CH

    mesh = plsc.VectorSubcoreMesh(
        core_axis_name="c", subcore_axis_name="s",
        num_cores=_NC, num_subcores=_NS)

    @functools.partial(
        pl.kernel,
        mesh=mesh,
        out_type=(
            jax.ShapeDtypeStruct((R, d), jnp.float32),
            jax.ShapeDtypeStruct((R, d), jnp.float32),
        ),
        scratch_types=[
            pltpu.VMEM((rpw,), jnp.int32),
            pltpu.VMEM((2, _CH, d), jnp.float32),
            pltpu.VMEM((2, _CH, d), jnp.float32),
            pltpu.SemaphoreType.DMA,
            pltpu.SemaphoreType.DMA,
            pltpu.SemaphoreType.DMA,
            pltpu.SemaphoreType.DMA,
        ],
    )
    def gather_kernel(k_hbm, v_hbm, idx_hbm, ko_hbm, vo_hbm,
                      idx_v, kbuf, vbuf, ksem0, ksem1, vsem0, vsem1):
        wid = lax.axis_index("s") * _NC + lax.axis_index("c")
        base = wid * rpw
        pltpu.sync_copy(idx_hbm.at[pl.ds(base, rpw)], idx_v)
        ksems = (ksem0, ksem1)
        vsems = (vsem0, vsem1)

        def start(c):
            p = c % 2
            ix = idx_v.at[pl.ds(c * _CH, _CH)]
            ck = pltpu.async_copy(k_hbm.at[ix], kbuf.at[p], ksems[p])
            cv = pltpu.async_copy(v_hbm.at[ix], vbuf.at[p], vsems[p])
            return ck, cv

        inflight = [start(0), start(1)]
        for c in range(n_chunks):
            p = c % 2
            ck, cv = inflight[p]
            ck.wait()
            cv.wait()
            pltpu.sync_copy(kbuf.at[p], ko_hbm.at[pl.ds(base + c * _CH, _CH)])
            pltpu.sync_copy(vbuf.at[p], vo_hbm.at[pl.ds(base + c * _CH, _CH)])
            if c + 2 < n_chunks:
                inflight[p] = start(c + 2)

    return gather_kernel(k2, v2, flat_idx)


def _attn_body(q_ref, k_ref, v_ref, o_ref, *, bg, sm_scale):
    ss = []
    for i in range(bg):
        qi = (q_ref[i] * sm_scale).astype(jnp.bfloat16)
        ki = k_ref[i].astype(jnp.bfloat16)
        ss.append(jax.lax.dot_general(
            qi, ki, (((1,), (1,)), ((), ())),
            preferred_element_type=jnp.float32))
    aa = []
    for s in ss:
        m = jnp.max(s, axis=1, keepdims=True)
        e = jnp.exp(s - m)
        l = jnp.sum(e, axis=1, keepdims=True)
        aa.append((e / l).astype(jnp.bfloat16))
    for i in range(bg):
        vi = v_ref[i].astype(jnp.bfloat16)
        o_ref[i] = jax.lax.dot_general(
            aa[i], vi, (((1,), (0,)), ((), ())),
            preferred_element_type=jnp.float32)


def _tc_attn(q_r, k_sel, v_sel, sm_scale):
    NG, G, d = q_r.shape
    L = k_sel.shape[1]
    BG = 16
    grid = (NG // BG,)
    return pl.pallas_call(
        functools.partial(_attn_body, bg=BG, sm_scale=sm_scale),
        grid=grid,
        in_specs=[
            pl.BlockSpec((BG, G, d), lambda i: (i, 0, 0)),
            pl.BlockSpec((BG, L, d), lambda i: (i, 0, 0)),
            pl.BlockSpec((BG, L, d), lambda i: (i, 0, 0)),
        ],
        out_specs=pl.BlockSpec((BG, G, d), lambda i: (i, 0, 0)),
        out_shape=jax.ShapeDtypeStruct((NG, G, d), jnp.float32),
    )(q_r, k_sel, v_sel)


def kernel(q, k, v, sparse_indices):
    B, S1, N1, d = q.shape
    _, S2, N2, _ = k.shape
    L = sparse_indices.shape[-1]
    G = N1 // N2
    R = B * S1 * N2 * L
    sm_scale = float(d) ** -0.5

    b_ix = jnp.arange(B, dtype=jnp.int32).reshape(B, 1, 1, 1)
    h_ix = jnp.arange(N2, dtype=jnp.int32).reshape(1, 1, N2, 1)
    flat = ((b_ix * S2 + sparse_indices.astype(jnp.int32)) * N2 + h_ix)
    flat = flat.reshape(R)

    k2 = k.reshape(B * S2 * N2, d)
    v2 = v.reshape(B * S2 * N2, d)

    q_r = q.reshape(B * S1 * N2, G, d)

    H = 2
    RH = R // H
    NGH = (B * S1 * N2) // H
    outs = []
    for h in range(H):
        k_sel, v_sel = _sc_gather(k2, v2, flat[h * RH:(h + 1) * RH], d)
        outs.append(_tc_attn(
            q_r[h * NGH:(h + 1) * NGH],
            k_sel.reshape(-1, L, d), v_sel.reshape(-1, L, d), sm_scale))
    out = jnp.concatenate(outs, axis=0)
    return out.reshape(B, S1, N1, d).astype(jnp.float16)


# speedup vs baseline: 4.8035x; 1.0687x over previous
import functools

import jax
import jax.numpy as jnp
from jax import lax
from jax.experimental import pallas as pl
from jax.experimental.pallas import tpu as pltpu
from jax.experimental.pallas import tpu_sc as plsc

_NC = 2
_NS = 16
_NW = _NC * _NS
_CH = 128


def _sc_gather(k2, v2, flat_idx, d):
    R = flat_idx.shape[0]
    rpw = R // _NW
    n_chunks = rpw // _CH

    mesh = plsc.VectorSubcoreMesh(
        core_axis_name="c", subcore_axis_name="s",
        num_cores=_NC, num_subcores=_NS)

    @functools.partial(
        pl.kernel,
        mesh=mesh,
        out_type=(
            jax.ShapeDtypeStruct((R, d), jnp.float32),
            jax.ShapeDtypeStruct((R, d), jnp.float32),
        ),
        scratch_types=[
            pltpu.VMEM((rpw,), jnp.int32),
            pltpu.VMEM((2, _CH, d), jnp.float32),
            pltpu.VMEM((2, _CH, d), jnp.float32),
            pltpu.SemaphoreType.DMA,
            pltpu.SemaphoreType.DMA,
            pltpu.SemaphoreType.DMA,
            pltpu.SemaphoreType.DMA,
        ],
    )
    def gather_kernel(k_hbm, v_hbm, idx_hbm, ko_hbm, vo_hbm,
                      idx_v, kbuf, vbuf, ksem0, ksem1, vsem0, vsem1):
        wid = lax.axis_index("s") * _NC + lax.axis_index("c")
        base = wid * rpw
        pltpu.sync_copy(idx_hbm.at[pl.ds(base, rpw)], idx_v)
        ksems = (ksem0, ksem1)
        vsems = (vsem0, vsem1)

        def start(c):
            p = c % 2
            ix = idx_v.at[pl.ds(c * _CH, _CH)]
            ck = pltpu.async_copy(k_hbm.at[ix], kbuf.at[p], ksems[p])
            cv = pltpu.async_copy(v_hbm.at[ix], vbuf.at[p], vsems[p])
            return ck, cv

        inflight = [start(0), start(1)]
        for c in range(n_chunks):
            p = c % 2
            ck, cv = inflight[p]
            ck.wait()
            cv.wait()
            pltpu.sync_copy(kbuf.at[p], ko_hbm.at[pl.ds(base + c * _CH, _CH)])
            pltpu.sync_copy(vbuf.at[p], vo_hbm.at[pl.ds(base + c * _CH, _CH)])
            if c + 2 < n_chunks:
                inflight[p] = start(c + 2)

    return gather_kernel(k2, v2, flat_idx)


def _attn_body(q_ref, k_ref, v_ref, o_ref, *, bg, sm_scale):
    ss = []
    for i in range(bg):
        qi = (q_ref[i] * sm_scale).astype(jnp.bfloat16)
        ki = k_ref[i].astype(jnp.bfloat16)
        ss.append(jax.lax.dot_general(
            qi, ki, (((1,), (1,)), ((), ())),
            preferred_element_type=jnp.float32))
    aa = []
    for s in ss:
        m = jnp.max(s, axis=1, keepdims=True)
        e = jnp.exp(s - m)
        l = jnp.sum(e, axis=1, keepdims=True)
        aa.append((e / l).astype(jnp.bfloat16))
    for i in range(bg):
        vi = v_ref[i].astype(jnp.bfloat16)
        o_ref[i] = jax.lax.dot_general(
            aa[i], vi, (((1,), (0,)), ((), ())),
            preferred_element_type=jnp.float32)


def _tc_attn(q_r, k_sel, v_sel, sm_scale):
    NG, G, d = q_r.shape
    L = k_sel.shape[1]
    BG = 32
    grid = (NG // BG,)
    return pl.pallas_call(
        functools.partial(_attn_body, bg=BG, sm_scale=sm_scale),
        grid=grid,
        in_specs=[
            pl.BlockSpec((BG, G, d), lambda i: (i, 0, 0)),
            pl.BlockSpec((BG, L, d), lambda i: (i, 0, 0)),
            pl.BlockSpec((BG, L, d), lambda i: (i, 0, 0)),
        ],
        out_specs=pl.BlockSpec((BG, G, d), lambda i: (i, 0, 0)),
        out_shape=jax.ShapeDtypeStruct((NG, G, d), jnp.float32),
    )(q_r, k_sel, v_sel)


def kernel(q, k, v, sparse_indices):
    B, S1, N1, d = q.shape
    _, S2, N2, _ = k.shape
    L = sparse_indices.shape[-1]
    G = N1 // N2
    R = B * S1 * N2 * L
    sm_scale = float(d) ** -0.5

    b_ix = jnp.arange(B, dtype=jnp.int32).reshape(B, 1, 1, 1)
    h_ix = jnp.arange(N2, dtype=jnp.int32).reshape(1, 1, N2, 1)
    flat = ((b_ix * S2 + sparse_indices.astype(jnp.int32)) * N2 + h_ix)
    flat = flat.reshape(R)

    k2 = k.reshape(B * S2 * N2, d)
    v2 = v.reshape(B * S2 * N2, d)

    q_r = q.reshape(B * S1 * N2, G, d)

    H = 2
    RH = R // H
    NGH = (B * S1 * N2) // H
    outs = []
    for h in range(H):
        k_sel, v_sel = _sc_gather(k2, v2, flat[h * RH:(h + 1) * RH], d)
        outs.append(_tc_attn(
            q_r[h * NGH:(h + 1) * NGH],
            k_sel.reshape(-1, L, d), v_sel.reshape(-1, L, d), sm_scale))
    out = jnp.concatenate(outs, axis=0)
    return out.reshape(B, S1, N1, d).astype(jnp.float16)


# speedup vs baseline: 4.9192x; 1.0241x over previous
import functools

import jax
import jax.numpy as jnp
from jax import lax
from jax.experimental import pallas as pl
from jax.experimental.pallas import tpu as pltpu
from jax.experimental.pallas import tpu_sc as plsc

_NC = 2
_NS = 16
_NW = _NC * _NS
_CH = 128


def _sc_gather(k2, v2, flat_idx, d):
    R = flat_idx.shape[0]
    rpw = R // _NW
    n_chunks = rpw // _CH

    mesh = plsc.VectorSubcoreMesh(
        core_axis_name="c", subcore_axis_name="s",
        num_cores=_NC, num_subcores=_NS)

    @functools.partial(
        pl.kernel,
        mesh=mesh,
        out_type=(
            jax.ShapeDtypeStruct((R, d), jnp.float32),
            jax.ShapeDtypeStruct((R, d), jnp.float32),
        ),
        scratch_types=[
            pltpu.VMEM((rpw,), jnp.int32),
            pltpu.VMEM((2, _CH, d), jnp.float32),
            pltpu.VMEM((2, _CH, d), jnp.float32),
            pltpu.SemaphoreType.DMA,
            pltpu.SemaphoreType.DMA,
            pltpu.SemaphoreType.DMA,
            pltpu.SemaphoreType.DMA,
        ],
    )
    def gather_kernel(k_hbm, v_hbm, idx_hbm, ko_hbm, vo_hbm,
                      idx_v, kbuf, vbuf, ksem0, ksem1, vsem0, vsem1):
        wid = lax.axis_index("s") * _NC + lax.axis_index("c")
        base = wid * rpw
        pltpu.sync_copy(idx_hbm.at[pl.ds(base, rpw)], idx_v)
        ksems = (ksem0, ksem1)
        vsems = (vsem0, vsem1)

        def start(c):
            p = c % 2
            ix = idx_v.at[pl.ds(c * _CH, _CH)]
            ck = pltpu.async_copy(k_hbm.at[ix], kbuf.at[p], ksems[p])
            cv = pltpu.async_copy(v_hbm.at[ix], vbuf.at[p], vsems[p])
            return ck, cv

        inflight = [start(0), start(1)]
        for c in range(n_chunks):
            p = c % 2
            ck, cv = inflight[p]
            ck.wait()
            cv.wait()
            pltpu.sync_copy(kbuf.at[p], ko_hbm.at[pl.ds(base + c * _CH, _CH)])
            pltpu.sync_copy(vbuf.at[p], vo_hbm.at[pl.ds(base + c * _CH, _CH)])
            if c + 2 < n_chunks:
                inflight[p] = start(c + 2)

    return gather_kernel(k2, v2, flat_idx)


def _attn_body(q_ref, k_ref, v_ref, o_ref, *, bg, sm_scale):
    ss = []
    for i in range(bg):
        qi = (q_ref[i] * sm_scale).astype(jnp.bfloat16)
        ki = k_ref[i].astype(jnp.bfloat16)
        ss.append(jax.lax.dot_general(
            qi, ki, (((1,), (1,)), ((), ())),
            preferred_element_type=jnp.float32))
    aa = []
    for s in ss:
        m = jnp.max(s, axis=1, keepdims=True)
        e = jnp.exp(s - m)
        l = jnp.sum(e, axis=1, keepdims=True)
        aa.append((e / l).astype(jnp.bfloat16))
    for i in range(bg):
        vi = v_ref[i].astype(jnp.bfloat16)
        o_ref[i] = jax.lax.dot_general(
            aa[i], vi, (((1,), (0,)), ((), ())),
            preferred_element_type=jnp.float32)


def _tc_attn(q_r, k_sel, v_sel, sm_scale):
    NG, G, d = q_r.shape
    L = k_sel.shape[1]
    BG = 64
    grid = (NG // BG,)
    return pl.pallas_call(
        functools.partial(_attn_body, bg=BG, sm_scale=sm_scale),
        grid=grid,
        in_specs=[
            pl.BlockSpec((BG, G, d), lambda i: (i, 0, 0)),
            pl.BlockSpec((BG, L, d), lambda i: (i, 0, 0)),
            pl.BlockSpec((BG, L, d), lambda i: (i, 0, 0)),
        ],
        out_specs=pl.BlockSpec((BG, G, d), lambda i: (i, 0, 0)),
        out_shape=jax.ShapeDtypeStruct((NG, G, d), jnp.float32),
    )(q_r, k_sel, v_sel)


def kernel(q, k, v, sparse_indices):
    B, S1, N1, d = q.shape
    _, S2, N2, _ = k.shape
    L = sparse_indices.shape[-1]
    G = N1 // N2
    R = B * S1 * N2 * L
    sm_scale = float(d) ** -0.5

    b_ix = jnp.arange(B, dtype=jnp.int32).reshape(B, 1, 1, 1)
    h_ix = jnp.arange(N2, dtype=jnp.int32).reshape(1, 1, N2, 1)
    flat = ((b_ix * S2 + sparse_indices.astype(jnp.int32)) * N2 + h_ix)
    flat = flat.reshape(R)

    k2 = k.reshape(B * S2 * N2, d)
    v2 = v.reshape(B * S2 * N2, d)

    q_r = q.reshape(B * S1 * N2, G, d)

    H = 2
    RH = R // H
    NGH = (B * S1 * N2) // H
    outs = []
    for h in range(H):
        k_sel, v_sel = _sc_gather(k2, v2, flat[h * RH:(h + 1) * RH], d)
        outs.append(_tc_attn(
            q_r[h * NGH:(h + 1) * NGH],
            k_sel.reshape(-1, L, d), v_sel.reshape(-1, L, d), sm_scale))
    out = jnp.concatenate(outs, axis=0)
    return out.reshape(B, S1, N1, d).astype(jnp.float16)
